# baseline (speedup 1.0000x reference)
import jax
import jax.numpy as jnp
import numpy as np
from jax import lax
from jax.experimental import pallas as pl
from jax.experimental.pallas import tpu as pltpu
import jax.experimental.pallas.tpu_sc as plsc

K = 8192
D = 32
N = 8192
BK = 256
NSTEP = K // BK
LANES = 128
SUB = BK // LANES
BIG = np.int32(2**30)


SUBL = 8
NGRP = BK // SUBL
WIN_STEPS = 2048 // BK
NWIN = NSTEP // WIN_STEPS


def _argmin_kernel(emb_ref, flat_ref, b_ref, a_ref, out_ref, loss_ref,
                   curv, curi, accv, acci, minve):
    k = pl.program_id(0)

    @pl.when(k == 0)
    def _init():
        accv[...] = jnp.full((1, N), jnp.inf, jnp.float32)
        acci[...] = jnp.zeros((1, N), jnp.int32)
        minve[...] = jnp.full((1, N), jnp.inf, jnp.float32)

    @pl.when(lax.rem(k, WIN_STEPS) == 0)
    def _reset():
        curv[...] = jnp.full((SUBL, N), jnp.inf, jnp.float32)
        curi[...] = jnp.zeros((SUBL, N), jnp.int32)

    m2 = lax.dot_general(
        emb_ref[...], flat_ref[...],
        dimension_numbers=(((1,), (1,)), ((), ())),
        preferred_element_type=jnp.float32,
    )
    ab = b_ref[...] + a_ref[...]
    dist = ab - m2

    for g in range(NGRP):
        d = dist[g * SUBL:(g + 1) * SUBL, :]
        v = curv[...]
        take = d < v
        curv[...] = jnp.where(take, d, v)
        curi[...] = jnp.where(take, k * NGRP + g, curi[...])

    @pl.when(lax.rem(k, WIN_STEPS) == WIN_STEPS - 1)
    def _fold_window():
        v = curv[...]
        i = curi[...]
        colmin = jnp.min(v, axis=0, keepdims=True)
        subl = lax.broadcasted_iota(jnp.int32, (SUBL, N), 0)
        full = i * SUBL + subl
        cand = jnp.where(v == colmin, full, BIG)
        win_i = jnp.min(cand, axis=0, keepdims=True)
        av = accv[...]
        ai = acci[...]
        keep = (av < colmin) | ((av == colmin) & (ai < win_i))
        newv = jnp.where(keep, av, colmin)
        accv[...] = newv.astype(jnp.bfloat16).astype(jnp.float32)
        acci[...] = jnp.where(keep, ai, win_i)
        minve[...] = jnp.minimum(minve[...], colmin)

    @pl.when(k == NSTEP - 1)
    def _finish():
        out_ref[...] = acci[...]
        s = jnp.sum(minve[...])
        loss_ref[...] = jnp.reshape(s * np.float32(1.25 / (N * D)), (1, 1))


def _run_argmin(flat, emb, a, b):
    return pl.pallas_call(
        _argmin_kernel,
        grid=(NSTEP,),
        in_specs=[
            pl.BlockSpec((BK, D), lambda k: (k, 0)),
            pl.BlockSpec((N, D), lambda k: (0, 0)),
            pl.BlockSpec((BK, 1), lambda k: (k, 0)),
            pl.BlockSpec((1, N), lambda k: (0, 0)),
        ],
        out_specs=(pl.BlockSpec((1, N), lambda k: (0, 0)),
                   pl.BlockSpec((1, 1), lambda k: (0, 0))),
        out_shape=(jax.ShapeDtypeStruct((1, N), jnp.int32),
                   jax.ShapeDtypeStruct((1, 1), jnp.float32)),
        scratch_shapes=[
            pltpu.VMEM((SUBL, N), jnp.float32),
            pltpu.VMEM((SUBL, N), jnp.int32),
            pltpu.VMEM((1, N), jnp.float32),
            pltpu.VMEM((1, N), jnp.int32),
            pltpu.VMEM((1, N), jnp.float32),
        ],
    )(emb, flat, b, a)


_NW = 32
_ROWS_PER_W = 2
_BINS_PER_W = K // _NW


def _sc_body(emb_hbm, idx_hbm, q_hbm, counts_hbm, idx_all, rows_v, counts_v, sem):
    nc = 2
    wid = lax.axis_index("s") * nc + lax.axis_index("c")

    pltpu.sync_copy(idx_hbm, idx_all)

    base = wid * _ROWS_PER_W
    copies = []
    for j in range(_ROWS_PER_W):
        copies.append(
            pltpu.async_copy(emb_hbm.at[idx_all.at[base + j]], rows_v.at[j], sem)
        )

    zeros16 = jnp.zeros((16,), jnp.float32)
    for i in range(_BINS_PER_W // 16):
        counts_v[pl.ds(i * 16, 16)] = zeros16
    lo = wid * _BINS_PER_W
    ones16 = jnp.full((16,), 1.0, jnp.float32)

    def row_body(r, carry):
        for i in range(LANES // 16):
            idx = idx_all[r, pl.ds(i * 16, 16)]
            rel = idx - lo
            mask = (rel >= 0) & (rel < _BINS_PER_W)
            rel = jnp.where(mask, rel, 0)
            plsc.addupdate_scatter(counts_v, [rel], ones16, mask=mask)
        return carry

    lax.fori_loop(0, N // LANES, row_body, 0)
    pltpu.sync_copy(counts_v, counts_hbm.at[pl.ds(lo, _BINS_PER_W)])

    for c in copies:
        c.wait()
    pltpu.sync_copy(rows_v, q_hbm.at[pl.ds(base, _ROWS_PER_W)])


def _run_sc(emb, idx2d):
    mesh = plsc.VectorSubcoreMesh(core_axis_name="c", subcore_axis_name="s")
    f = pl.kernel(
        _sc_body,
        out_type=(
            jax.ShapeDtypeStruct((N // LANES, LANES, D), jnp.float32),
            jax.ShapeDtypeStruct((K,), jnp.float32),
        ),
        mesh=mesh,
        compiler_params=pltpu.CompilerParams(
            needs_layout_passes=False, use_tc_tiling_on_sc=False),
        scratch_types=[
            pltpu.VMEM((N // LANES, LANES), jnp.int32),
            pltpu.VMEM((_ROWS_PER_W, LANES, D), jnp.float32),
            pltpu.VMEM((_BINS_PER_W,), jnp.float32),
            pltpu.SemaphoreType.DMA,
        ],
    )
    return f(emb, idx2d)


def _scalars_kernel(c_ref, perp_ref):
    p = c_ref[...] * np.float32(1.0 / N)
    ent = p * jnp.log(p + np.float32(1e-10))
    perp_ref[...] = jnp.reshape(jnp.exp(-jnp.sum(ent)), (1, 1))


def _run_scalars(counts2d):
    return pl.pallas_call(
        _scalars_kernel,
        out_shape=jax.ShapeDtypeStruct((1, 1), jnp.float32),
    )(counts2d)


@jax.jit
def kernel(latents, embedding):
    lat = jnp.transpose(latents, (0, 2, 3, 4, 1))
    shape5 = lat.shape
    flat = lat.reshape(-1, D)
    a = jnp.sum(flat ** 2, axis=1).reshape(1, N)
    embT = lax.optimization_barrier(embedding.T)
    b = jnp.sum(embT ** 2, axis=0).reshape(K, 1)

    emb2 = embedding + embedding
    idx_b, loss = _run_argmin(flat, emb2, a, b)
    inds = idx_b.reshape(N)
    idx2d = inds.reshape(N // LANES, LANES)

    q3, counts = _run_sc(embedding, idx2d)
    q = q3.reshape(N, D)

    perp = _run_scalars(counts.reshape(N // LANES, LANES))

    out = jnp.transpose(q.reshape(shape5), (0, 4, 1, 2, 3))
    return (out, loss[0, 0], inds, perp[0, 0])


# speedup vs baseline: 1.0616x; 1.0616x over previous
import jax
import jax.numpy as jnp
import numpy as np
from jax import lax
from jax.experimental import pallas as pl
from jax.experimental.pallas import tpu as pltpu
import jax.experimental.pallas.tpu_sc as plsc

K = 8192
D = 32
N = 8192
BK = 1024
NSTEP = K // BK
LANES = 128
SUB = BK // LANES
BIG = np.int32(2**30)


SUBL = 8
NGRP = BK // SUBL
WIN_STEPS = 2048 // BK
NWIN = NSTEP // WIN_STEPS


def _argmin_kernel(emb_ref, flat_ref, b_ref, a_ref, out_ref, loss_ref,
                   curv, curi, accv, acci, minve):
    k = pl.program_id(0)

    @pl.when(k == 0)
    def _init():
        accv[...] = jnp.full((1, N), jnp.inf, jnp.float32)
        acci[...] = jnp.zeros((1, N), jnp.int32)
        minve[...] = jnp.full((1, N), jnp.inf, jnp.float32)

    @pl.when(lax.rem(k, WIN_STEPS) == 0)
    def _reset():
        curv[...] = jnp.full((SUBL, N), jnp.inf, jnp.float32)
        curi[...] = jnp.zeros((SUBL, N), jnp.int32)

    m2 = lax.dot_general(
        emb_ref[...], flat_ref[...],
        dimension_numbers=(((1,), (1,)), ((), ())),
        preferred_element_type=jnp.float32,
    )
    ab = b_ref[...] + a_ref[...]
    dist = ab - m2

    for g in range(NGRP):
        d = dist[g * SUBL:(g + 1) * SUBL, :]
        v = curv[...]
        take = d < v
        curv[...] = jnp.where(take, d, v)
        curi[...] = jnp.where(take, k * NGRP + g, curi[...])

    @pl.when(lax.rem(k, WIN_STEPS) == WIN_STEPS - 1)
    def _fold_window():
        v = curv[...]
        i = curi[...]
        colmin = jnp.min(v, axis=0, keepdims=True)
        subl = lax.broadcasted_iota(jnp.int32, (SUBL, N), 0)
        full = i * SUBL + subl
        cand = jnp.where(v == colmin, full, BIG)
        win_i = jnp.min(cand, axis=0, keepdims=True)
        av = accv[...]
        ai = acci[...]
        keep = (av < colmin) | ((av == colmin) & (ai < win_i))
        newv = jnp.where(keep, av, colmin)
        accv[...] = newv.astype(jnp.bfloat16).astype(jnp.float32)
        acci[...] = jnp.where(keep, ai, win_i)
        minve[...] = jnp.minimum(minve[...], colmin)

    @pl.when(k == NSTEP - 1)
    def _finish():
        out_ref[...] = acci[...]
        s = jnp.sum(minve[...])
        loss_ref[...] = jnp.reshape(s * np.float32(1.25 / (N * D)), (1, 1))


def _run_argmin(flat, emb, a, b):
    return pl.pallas_call(
        _argmin_kernel,
        grid=(NSTEP,),
        in_specs=[
            pl.BlockSpec((BK, D), lambda k: (k, 0)),
            pl.BlockSpec((N, D), lambda k: (0, 0)),
            pl.BlockSpec((BK, 1), lambda k: (k, 0)),
            pl.BlockSpec((1, N), lambda k: (0, 0)),
        ],
        out_specs=(pl.BlockSpec((1, N), lambda k: (0, 0)),
                   pl.BlockSpec((1, 1), lambda k: (0, 0))),
        out_shape=(jax.ShapeDtypeStruct((1, N), jnp.int32),
                   jax.ShapeDtypeStruct((1, 1), jnp.float32)),
        scratch_shapes=[
            pltpu.VMEM((SUBL, N), jnp.float32),
            pltpu.VMEM((SUBL, N), jnp.int32),
            pltpu.VMEM((1, N), jnp.float32),
            pltpu.VMEM((1, N), jnp.int32),
            pltpu.VMEM((1, N), jnp.float32),
        ],
    )(emb, flat, b, a)


_NW = 32
_ROWS_PER_W = 2
_BINS_PER_W = K // _NW


def _sc_body(emb_hbm, idx_hbm, q_hbm, counts_hbm, idx_all, rows_v, counts_v, sem):
    nc = 2
    wid = lax.axis_index("s") * nc + lax.axis_index("c")

    pltpu.sync_copy(idx_hbm, idx_all)

    base = wid * _ROWS_PER_W
    copies = []
    for j in range(_ROWS_PER_W):
        copies.append(
            pltpu.async_copy(emb_hbm.at[idx_all.at[base + j]], rows_v.at[j], sem)
        )

    zeros16 = jnp.zeros((16,), jnp.float32)
    for i in range(_BINS_PER_W // 16):
        counts_v[pl.ds(i * 16, 16)] = zeros16
    lo = wid * _BINS_PER_W
    ones16 = jnp.full((16,), 1.0, jnp.float32)

    def row_body(r, carry):
        for i in range(LANES // 16):
            idx = idx_all[r, pl.ds(i * 16, 16)]
            rel = idx - lo
            mask = (rel >= 0) & (rel < _BINS_PER_W)
            rel = jnp.where(mask, rel, 0)
            plsc.addupdate_scatter(counts_v, [rel], ones16, mask=mask)
        return carry

    lax.fori_loop(0, N // LANES, row_body, 0)
    pltpu.sync_copy(counts_v, counts_hbm.at[pl.ds(lo, _BINS_PER_W)])

    for c in copies:
        c.wait()
    pltpu.sync_copy(rows_v, q_hbm.at[pl.ds(base, _ROWS_PER_W)])


def _run_sc(emb, idx2d):
    mesh = plsc.VectorSubcoreMesh(core_axis_name="c", subcore_axis_name="s")
    f = pl.kernel(
        _sc_body,
        out_type=(
            jax.ShapeDtypeStruct((N // LANES, LANES, D), jnp.float32),
            jax.ShapeDtypeStruct((K,), jnp.float32),
        ),
        mesh=mesh,
        compiler_params=pltpu.CompilerParams(
            needs_layout_passes=False, use_tc_tiling_on_sc=False),
        scratch_types=[
            pltpu.VMEM((N // LANES, LANES), jnp.int32),
            pltpu.VMEM((_ROWS_PER_W, LANES, D), jnp.float32),
            pltpu.VMEM((_BINS_PER_W,), jnp.float32),
            pltpu.SemaphoreType.DMA,
        ],
    )
    return f(emb, idx2d)


def _scalars_kernel(c_ref, perp_ref):
    p = c_ref[...] * np.float32(1.0 / N)
    ent = p * jnp.log(p + np.float32(1e-10))
    perp_ref[...] = jnp.reshape(jnp.exp(-jnp.sum(ent)), (1, 1))


def _run_scalars(counts2d):
    return pl.pallas_call(
        _scalars_kernel,
        out_shape=jax.ShapeDtypeStruct((1, 1), jnp.float32),
    )(counts2d)


@jax.jit
def kernel(latents, embedding):
    lat = jnp.transpose(latents, (0, 2, 3, 4, 1))
    shape5 = lat.shape
    flat = lat.reshape(-1, D)
    a = jnp.sum(flat ** 2, axis=1).reshape(1, N)
    embT = lax.optimization_barrier(embedding.T)
    b = jnp.sum(embT ** 2, axis=0).reshape(K, 1)

    emb2 = embedding + embedding
    idx_b, loss = _run_argmin(flat, emb2, a, b)
    inds = idx_b.reshape(N)
    idx2d = inds.reshape(N // LANES, LANES)

    q3, counts = _run_sc(embedding, idx2d)
    q = q3.reshape(N, D)

    perp = _run_scalars(counts.reshape(N // LANES, LANES))

    out = jnp.transpose(q.reshape(shape5), (0, 4, 1, 2, 3))
    return (out, loss[0, 0], inds, perp[0, 0])


# speedup vs baseline: 1.0875x; 1.0244x over previous
import jax
import jax.numpy as jnp
import numpy as np
from jax import lax
from jax.experimental import pallas as pl
from jax.experimental.pallas import tpu as pltpu
import jax.experimental.pallas.tpu_sc as plsc

K = 8192
D = 32
N = 8192
BK = 2048
NSTEP = K // BK
LANES = 128
SUB = BK // LANES
BIG = np.int32(2**30)


SUBL = 8
NGRP = BK // SUBL
WIN_STEPS = 2048 // BK
NWIN = NSTEP // WIN_STEPS


def _argmin_kernel(emb_ref, flat_ref, b_ref, a_ref, out_ref, loss_ref,
                   curv, curi, accv, acci, minve):
    k = pl.program_id(0)

    @pl.when(k == 0)
    def _init():
        accv[...] = jnp.full((1, N), jnp.inf, jnp.float32)
        acci[...] = jnp.zeros((1, N), jnp.int32)
        minve[...] = jnp.full((1, N), jnp.inf, jnp.float32)

    @pl.when(lax.rem(k, WIN_STEPS) == 0)
    def _reset():
        curv[...] = jnp.full((SUBL, N), jnp.inf, jnp.float32)
        curi[...] = jnp.zeros((SUBL, N), jnp.int32)

    m2 = lax.dot_general(
        emb_ref[...], flat_ref[...],
        dimension_numbers=(((1,), (1,)), ((), ())),
        preferred_element_type=jnp.float32,
    )
    ab = b_ref[...] + a_ref[...]
    dist = ab - m2

    for g in range(NGRP):
        d = dist[g * SUBL:(g + 1) * SUBL, :]
        v = curv[...]
        take = d < v
        curv[...] = jnp.where(take, d, v)
        curi[...] = jnp.where(take, k * NGRP + g, curi[...])

    @pl.when(lax.rem(k, WIN_STEPS) == WIN_STEPS - 1)
    def _fold_window():
        v = curv[...]
        i = curi[...]
        colmin = jnp.min(v, axis=0, keepdims=True)
        subl = lax.broadcasted_iota(jnp.int32, (SUBL, N), 0)
        full = i * SUBL + subl
        cand = jnp.where(v == colmin, full, BIG)
        win_i = jnp.min(cand, axis=0, keepdims=True)
        av = accv[...]
        ai = acci[...]
        keep = (av < colmin) | ((av == colmin) & (ai < win_i))
        newv = jnp.where(keep, av, colmin)
        accv[...] = newv.astype(jnp.bfloat16).astype(jnp.float32)
        acci[...] = jnp.where(keep, ai, win_i)
        minve[...] = jnp.minimum(minve[...], colmin)

    @pl.when(k == NSTEP - 1)
    def _finish():
        out_ref[...] = acci[...]
        s = jnp.sum(minve[...])
        loss_ref[...] = jnp.reshape(s * np.float32(1.25 / (N * D)), (1, 1))


def _run_argmin(flat, emb, a, b):
    return pl.pallas_call(
        _argmin_kernel,
        grid=(NSTEP,),
        in_specs=[
            pl.BlockSpec((BK, D), lambda k: (k, 0)),
            pl.BlockSpec((N, D), lambda k: (0, 0)),
            pl.BlockSpec((BK, 1), lambda k: (k, 0)),
            pl.BlockSpec((1, N), lambda k: (0, 0)),
        ],
        out_specs=(pl.BlockSpec((1, N), lambda k: (0, 0)),
                   pl.BlockSpec((1, 1), lambda k: (0, 0))),
        out_shape=(jax.ShapeDtypeStruct((1, N), jnp.int32),
                   jax.ShapeDtypeStruct((1, 1), jnp.float32)),
        scratch_shapes=[
            pltpu.VMEM((SUBL, N), jnp.float32),
            pltpu.VMEM((SUBL, N), jnp.int32),
            pltpu.VMEM((1, N), jnp.float32),
            pltpu.VMEM((1, N), jnp.int32),
            pltpu.VMEM((1, N), jnp.float32),
        ],
    )(emb, flat, b, a)


_NW = 32
_ROWS_PER_W = 2
_BINS_PER_W = K // _NW


def _sc_body(emb_hbm, idx_hbm, q_hbm, counts_hbm, idx_all, rows_v, counts_v, sem):
    nc = 2
    wid = lax.axis_index("s") * nc + lax.axis_index("c")

    pltpu.sync_copy(idx_hbm, idx_all)

    base = wid * _ROWS_PER_W
    copies = []
    for j in range(_ROWS_PER_W):
        copies.append(
            pltpu.async_copy(emb_hbm.at[idx_all.at[base + j]], rows_v.at[j], sem)
        )

    zeros16 = jnp.zeros((16,), jnp.float32)
    for i in range(_BINS_PER_W // 16):
        counts_v[pl.ds(i * 16, 16)] = zeros16
    lo = wid * _BINS_PER_W
    ones16 = jnp.full((16,), 1.0, jnp.float32)

    def row_body(r, carry):
        for i in range(LANES // 16):
            idx = idx_all[r, pl.ds(i * 16, 16)]
            rel = idx - lo
            mask = (rel >= 0) & (rel < _BINS_PER_W)
            rel = jnp.where(mask, rel, 0)
            plsc.addupdate_scatter(counts_v, [rel], ones16, mask=mask)
        return carry

    lax.fori_loop(0, N // LANES, row_body, 0)
    pltpu.sync_copy(counts_v, counts_hbm.at[pl.ds(lo, _BINS_PER_W)])

    for c in copies:
        c.wait()
    pltpu.sync_copy(rows_v, q_hbm.at[pl.ds(base, _ROWS_PER_W)])


def _run_sc(emb, idx2d):
    mesh = plsc.VectorSubcoreMesh(core_axis_name="c", subcore_axis_name="s")
    f = pl.kernel(
        _sc_body,
        out_type=(
            jax.ShapeDtypeStruct((N // LANES, LANES, D), jnp.float32),
            jax.ShapeDtypeStruct((K,), jnp.float32),
        ),
        mesh=mesh,
        compiler_params=pltpu.CompilerParams(
            needs_layout_passes=False, use_tc_tiling_on_sc=False),
        scratch_types=[
            pltpu.VMEM((N // LANES, LANES), jnp.int32),
            pltpu.VMEM((_ROWS_PER_W, LANES, D), jnp.float32),
            pltpu.VMEM((_BINS_PER_W,), jnp.float32),
            pltpu.SemaphoreType.DMA,
        ],
    )
    return f(emb, idx2d)


def _scalars_kernel(c_ref, perp_ref):
    p = c_ref[...] * np.float32(1.0 / N)
    ent = p * jnp.log(p + np.float32(1e-10))
    perp_ref[...] = jnp.reshape(jnp.exp(-jnp.sum(ent)), (1, 1))


def _run_scalars(counts2d):
    return pl.pallas_call(
        _scalars_kernel,
        out_shape=jax.ShapeDtypeStruct((1, 1), jnp.float32),
    )(counts2d)


@jax.jit
def kernel(latents, embedding):
    lat = jnp.transpose(latents, (0, 2, 3, 4, 1))
    shape5 = lat.shape
    flat = lat.reshape(-1, D)
    a = jnp.sum(flat ** 2, axis=1).reshape(1, N)
    embT = lax.optimization_barrier(embedding.T)
    b = jnp.sum(embT ** 2, axis=0).reshape(K, 1)

    emb2 = embedding + embedding
    idx_b, loss = _run_argmin(flat, emb2, a, b)
    inds = idx_b.reshape(N)
    idx2d = inds.reshape(N // LANES, LANES)

    q3, counts = _run_sc(embedding, idx2d)
    q = q3.reshape(N, D)

    perp = _run_scalars(counts.reshape(N // LANES, LANES))

    out = jnp.transpose(q.reshape(shape5), (0, 4, 1, 2, 3))
    return (out, loss[0, 0], inds, perp[0, 0])
